# baseline (speedup 1.0000x reference)
import functools

import jax
import jax.numpy as jnp
from jax import lax
from jax.experimental import pallas as pl
from jax.experimental.pallas import tpu as pltpu
from jax.experimental.pallas import tpu_sc as plsc

CLS = 100
CLS_PAD = 128
FEAT = 128
BATCH = 16384
LAM = 0.1

NC = 2
NS = 16
NW = NC * NS
B_PER_W = BATCH // NW
B_BIN = BATCH // NS

_mesh = plsc.VectorSubcoreMesh(core_axis_name="c", subcore_axis_name="s")


@functools.partial(
    pl.kernel,
    out_type=jax.ShapeDtypeStruct((NW, 16), jnp.float32),
    mesh=_mesh,
    compiler_params=pltpu.CompilerParams(needs_layout_passes=False),
    scratch_types=[
        pltpu.VMEM((B_PER_W * FEAT,), jnp.float32),
        pltpu.VMEM((B_PER_W + 16,), jnp.int32),
        pltpu.VMEM((B_BIN + 16,), jnp.int32),
        pltpu.VMEM((B_BIN,), jnp.float32),
        pltpu.VMEM((CLS * FEAT,), jnp.float32),
        pltpu.VMEM((CLS_PAD,), jnp.float32),
        pltpu.VMEM((NS, CLS_PAD), jnp.float32),
        pltpu.VMEM_SHARED((NS, CLS_PAD), jnp.float32),
        pltpu.VMEM((16,), jnp.float32),
        pltpu.SemaphoreType.DMA,
    ],
)
def _anchor_loss_kernel(feat_hbm, tgt_hbm, anc_hbm, out_hbm,
                        feat_v, idx_v, bin_v, ftgt_v, anc_v, wts_v, sums_v,
                        shared_cnt, out_v, sem):
    sid = lax.axis_index("s")
    wid = sid * NC + lax.axis_index("c")
    base = wid * B_PER_W

    fcopy = pltpu.async_copy(
        feat_hbm.at[pl.ds(base * FEAT, B_PER_W * FEAT)], feat_v, sem)

    pltpu.sync_copy(tgt_hbm.at[pl.ds(sid * B_BIN, B_BIN)], ftgt_v)
    pltpu.sync_copy(anc_hbm, anc_v)
    for g in range(B_BIN // 16):
        bin_v[pl.ds(16 * g, 16)] = ftgt_v[pl.ds(16 * g, 16)].astype(jnp.int32)
    pltpu.sync_copy(tgt_hbm.at[pl.ds(base, B_PER_W)],
                    ftgt_v.at[pl.ds(0, B_PER_W)])
    for g in range(B_PER_W // 16):
        idx_v[pl.ds(16 * g, 16)] = ftgt_v[pl.ds(16 * g, 16)].astype(jnp.int32)

    for c in range(CLS_PAD // 16):
        wts_v[pl.ds(16 * c, 16)] = jnp.zeros((16,), jnp.float32)
    lane0 = lax.iota(jnp.int32, 16) == 0
    ones = jnp.ones((16,), jnp.float32)

    def bin_body(s, carry):
        t = bin_v[pl.ds(s, 16)][0]
        tvec = jnp.full((16,), t, jnp.int32)
        plsc.addupdate_scatter(wts_v, [tvec], ones, mask=lane0)
        return carry

    lax.fori_loop(0, B_BIN, bin_body, 0, unroll=4)

    pltpu.sync_copy(wts_v, shared_cnt.at[sid])
    plsc.subcore_barrier()
    pltpu.sync_copy(shared_cnt, sums_v)
    for c in range(CLS_PAD // 16):
        tot = sums_v[0, pl.ds(16 * c, 16)]
        for r in range(1, NS):
            tot = tot + sums_v[r, pl.ds(16 * c, 16)]
        w = jnp.where(tot > 0.0, 1.0 / tot, 0.0)
        wts_v[pl.ds(16 * c, 16)] = w

    fcopy.wait()

    def body(s, grand):
        t = idx_v[pl.ds(s, 16)][0]
        w = wts_v[pl.ds(t, 16)][0]
        frow = s * FEAT
        arow = t * FEAT
        acc = None
        for c in range(FEAT // 16):
            f = feat_v[pl.ds(frow + 16 * c, 16)]
            a = anc_v[pl.ds(arow + 16 * c, 16)]
            d = f - a
            p = d * d
            acc = p if acc is None else acc + p
        return grand + w * acc

    grand = lax.fori_loop(0, B_PER_W, body, jnp.zeros((16,), jnp.float32),
                          unroll=4)
    out_v[...] = grand
    pltpu.sync_copy(out_v, out_hbm.at[wid])


def kernel(feature, _target, anchor):
    partials = _anchor_loss_kernel(
        feature.reshape(-1), _target, anchor.reshape(-1))
    return (LAM / CLS) * jnp.sum(partials)


# speedup vs baseline: 1.2578x; 1.2578x over previous
import functools

import jax
import jax.numpy as jnp
from jax import lax
from jax.experimental import pallas as pl
from jax.experimental.pallas import tpu as pltpu
from jax.experimental.pallas import tpu_sc as plsc

CLS = 100
CLS_PAD = 128
FEAT = 128
BATCH = 16384
LAM = 0.1

NC = 2
NS = 16
NW = NC * NS
B_BIN = BATCH // NS

NBLK = 8
BLK = BATCH // NBLK


@functools.partial(
    pl.kernel,
    out_type=jax.ShapeDtypeStruct((NW, CLS_PAD), jnp.float32),
    mesh=plsc.VectorSubcoreMesh(core_axis_name="c", subcore_axis_name="s"),
    compiler_params=pltpu.CompilerParams(needs_layout_passes=False),
    scratch_types=[
        pltpu.VMEM((B_BIN + 16,), jnp.int32),
        pltpu.VMEM((B_BIN,), jnp.float32),
        pltpu.VMEM((CLS_PAD,), jnp.float32),
        pltpu.VMEM((NS, CLS_PAD), jnp.float32),
        pltpu.VMEM_SHARED((NS, CLS_PAD), jnp.float32),
        pltpu.SemaphoreType.DMA,
    ],
)
def _count_kernel(tgt_hbm, out_hbm, bin_v, fbin_v, cnt_v, sums_v,
                  shared_cnt, semb):
    sid = lax.axis_index("s")
    wid = sid * NC + lax.axis_index("c")

    bcopy = pltpu.async_copy(tgt_hbm.at[pl.ds(sid * B_BIN, B_BIN)],
                             fbin_v, semb)
    for c in range(CLS_PAD // 16):
        cnt_v[pl.ds(16 * c, 16)] = jnp.zeros((16,), jnp.float32)
    bcopy.wait()
    for g in range(B_BIN // 16):
        bin_v[pl.ds(16 * g, 16)] = fbin_v[pl.ds(16 * g, 16)].astype(jnp.int32)

    def bin_body(g, carry):
        t16 = bin_v[pl.ds(16 * g, 16)]
        cnt, last = plsc.scan_count(t16)
        plsc.addupdate_scatter(cnt_v, [t16], cnt.astype(jnp.float32),
                               mask=last)
        return carry

    lax.fori_loop(0, B_BIN // 16, bin_body, 0, unroll=4)

    pltpu.sync_copy(cnt_v, shared_cnt.at[sid])
    plsc.subcore_barrier()
    pltpu.sync_copy(shared_cnt, sums_v)
    for c in range(CLS_PAD // 16):
        tot = sums_v[0, pl.ds(16 * c, 16)]
        for r in range(1, NS):
            tot = tot + sums_v[r, pl.ds(16 * c, 16)]
        cnt_v[pl.ds(16 * c, 16)] = tot
    pltpu.sync_copy(cnt_v, out_hbm.at[wid])


def _tc_body(feat_ref, tgt_ref, anc_ref, out_ref, s_acc, ns_acc):
    i = pl.program_id(0)

    @pl.when(i == 0)
    def _init():
        s_acc[...] = jnp.zeros_like(s_acc)
        ns_acc[...] = jnp.zeros_like(ns_acc)

    f = feat_ref[0]
    t = tgt_ref[0]
    cls_iota = lax.broadcasted_iota(jnp.int32, (BLK, CLS_PAD), 1)
    onehot = (t.reshape(BLK, 1).astype(jnp.int32) == cls_iota).astype(
        jnp.float32)
    s_acc[...] += lax.dot_general(
        onehot, f, (((0,), (0,)), ((), ())),
        preferred_element_type=jnp.float32)
    n = jnp.sum(f * f, axis=1, keepdims=True)
    ns_acc[...] += lax.dot_general(
        n, onehot, (((0,), (0,)), ((), ())),
        preferred_element_type=jnp.float32)

    @pl.when(i == NBLK - 1)
    def _fin():
        anc = anc_ref[...]
        out_ref[0, :] = ns_acc[0, :]
        out_ref[1, :] = jnp.sum(s_acc[...] * anc, axis=1)
        out_ref[2, :] = jnp.sum(anc * anc, axis=1)


_tc_kernel = pl.pallas_call(
    _tc_body,
    grid=(NBLK,),
    in_specs=[
        pl.BlockSpec((1, BLK, FEAT), lambda i: (i, 0, 0)),
        pl.BlockSpec((1, 1, BLK), lambda i: (i, 0, 0)),
        pl.BlockSpec((CLS_PAD, FEAT), lambda i: (0, 0)),
    ],
    out_specs=pl.BlockSpec((3, CLS_PAD), lambda i: (0, 0)),
    out_shape=jax.ShapeDtypeStruct((3, CLS_PAD), jnp.float32),
    scratch_shapes=[
        pltpu.VMEM((CLS_PAD, FEAT), jnp.float32),
        pltpu.VMEM((1, CLS_PAD), jnp.float32),
    ],
    compiler_params=pltpu.CompilerParams(
        dimension_semantics=("arbitrary",)),
)


def kernel(feature, _target, anchor):
    counts = _count_kernel(_target)[0]
    anc = jnp.pad(anchor, ((0, CLS_PAD - CLS), (0, 0)))
    terms = _tc_kernel(
        feature.reshape(NBLK, BLK, FEAT),
        _target.reshape(NBLK, 1, BLK),
        anc)
    wts = jnp.where(counts > 0, 1.0 / counts, 0.0)
    ns, dotv, na = terms[0], terms[1], terms[2]
    loss = (jnp.sum(ns * wts) - 2.0 * jnp.sum(dotv * wts)
            + jnp.sum(jnp.where(counts > 0, na, 0.0)))
    return (LAM / CLS) * loss
